# baseline (speedup 1.0000x reference)
import functools

import jax
import jax.numpy as jnp
from jax import lax
from jax.experimental import pallas as pl
from jax.experimental.pallas import tpu as pltpu, tpu_sc as plsc

BATCH = 16384
FACTORS = 32

_info = plsc.get_sparse_core_info()
_NC, _NS, _L = _info.num_cores, _info.num_subcores, _info.num_lanes
_NW = _NC * _NS
_BPW = BATCH // _NW
_GROUPS = _BPW // _L

_mesh = plsc.VectorSubcoreMesh(core_axis_name="c", subcore_axis_name="s")


@functools.partial(
    pl.kernel,
    mesh=_mesh,
    out_type=jax.ShapeDtypeStruct((BATCH,), jnp.float32),
    compiler_params=pltpu.CompilerParams(needs_layout_passes=False,
                                         use_tc_tiling_on_sc=False),
    scratch_types=[
        pltpu.VMEM((_BPW,), jnp.int32),
        pltpu.VMEM((_BPW,), jnp.int32),
        pltpu.VMEM((_BPW, FACTORS), jnp.float32),
        pltpu.VMEM((_BPW, FACTORS), jnp.float32),
        pltpu.VMEM((_BPW,), jnp.float32),
        pltpu.VMEM((_BPW,), jnp.float32),
        pltpu.VMEM((_L,), jnp.float32),
        pltpu.VMEM((_BPW,), jnp.float32),
        pltpu.SemaphoreType.DMA,
    ],
)
def _mf_kernel(user_hbm, item_hbm, u_tab, i_tab, bu_tab, bi_tab, gb_hbm,
               out_hbm,
               uidx_v, iidx_v, urows_v, irows_v, ub_v, ib_v, gb_v, out_v,
               sem):
    wid = lax.axis_index("s") * _NC + lax.axis_index("c")
    base = wid * _BPW

    pltpu.sync_copy(user_hbm.at[pl.ds(base, _BPW)], uidx_v)
    pltpu.sync_copy(item_hbm.at[pl.ds(base, _BPW)], iidx_v)
    pltpu.sync_copy(gb_hbm, gb_v)

    cp_u = pltpu.async_copy(u_tab.at[uidx_v], urows_v, sem)
    cp_i = pltpu.async_copy(i_tab.at[iidx_v], irows_v, sem)
    cp_bu = pltpu.async_copy(bu_tab.at[uidx_v], ub_v, sem)
    cp_bi = pltpu.async_copy(bi_tab.at[iidx_v], ib_v, sem)
    cp_u.wait()
    cp_i.wait()
    cp_bu.wait()
    cp_bi.wait()

    gb = gb_v[...]
    lanes = lax.iota(jnp.int32, _L)
    cols = [jnp.full((_L,), k, jnp.int32) for k in range(FACTORS)]

    def body(g, carry):
        row0 = g * _L
        ridx = row0 + lanes
        accs = [jnp.zeros((_L,), jnp.float32) for _ in range(4)]
        for k in range(FACTORS):
            u = plsc.load_gather(urows_v, [ridx, cols[k]])
            v = plsc.load_gather(irows_v, [ridx, cols[k]])
            accs[k % 4] = accs[k % 4] + u * v
        acc = (accs[0] + accs[1]) + (accs[2] + accs[3])
        ub = ub_v[pl.ds(row0, _L)]
        ib = ib_v[pl.ds(row0, _L)]
        out_v[pl.ds(row0, _L)] = acc + ub + ib + gb
        return carry

    lax.fori_loop(0, _GROUPS, body, 0)

    pltpu.sync_copy(out_v, out_hbm.at[pl.ds(base, _BPW)])


def kernel(user, item, U_MF, I_MF, B_U, B_I, GB):
    bu_flat = B_U.reshape(-1)
    bi_flat = B_I.reshape(-1)
    gb_vec = jnp.broadcast_to(GB.astype(jnp.float32).reshape(1), (_L,))
    return _mf_kernel(user.astype(jnp.int32), item.astype(jnp.int32),
                      U_MF, I_MF, bu_flat, bi_flat, gb_vec)
